# baseline (speedup 1.0000x reference)
import functools
import math

import jax
import jax.numpy as jnp
from jax import lax
from jax.experimental import pallas as pl
from jax.experimental.pallas import tpu as pltpu
from jax.experimental.pallas import tpu_sc as plsc

_NC = 2
_NS = 16

_SC_ROWS = 4608
_CH = 16


def _sc_reduce_body(scores_ref, hid_ref, out_ref, ftok_ref, buf0, buf1, acc,
                    sem0, sem1, *, row_base, spw, ch, nl):
    c = lax.axis_index("c")
    s = lax.axis_index("s")
    row0 = row_base + s * spw
    nch = spw // ch
    zero16 = jnp.zeros((16,), jnp.float32)

    def zbody(j, _):
        acc[0, pl.ds(j * 16, 16)] = zero16
        return 0

    lax.fori_loop(0, nl // 16, zbody, 0)

    def copy_in(i, buf, sem):
        return pltpu.async_copy(
            scores_ref.at[c, pl.ds(row0 + i * ch, ch)], buf, sem)

    def wait_in(buf, sem):
        pltpu.make_async_copy(scores_ref.at[c, pl.ds(row0, ch)], buf, sem).wait()

    def accum(buf):
        def jbody(j, _):
            off = j * 16

            def rbody(r4, vs):
                v0, v1, v2, v3 = vs
                r = r4 * 4
                v0 = v0 + buf[r, pl.ds(off, 16)]
                v1 = v1 + buf[r + 1, pl.ds(off, 16)]
                v2 = v2 + buf[r + 2, pl.ds(off, 16)]
                v3 = v3 + buf[r + 3, pl.ds(off, 16)]
                return (v0, v1, v2, v3)

            v0, v1, v2, v3 = lax.fori_loop(
                0, ch // 4, rbody, (zero16, zero16, zero16, zero16), unroll=2)
            plsc.addupdate(acc.at[0, pl.ds(off, 16)], (v0 + v1) + (v2 + v3))
            return 0

        lax.fori_loop(0, nl // 16, jbody, 0, unroll=2)

    copy_in(0, buf0, sem0)

    def body(i2, _):
        i = i2 * 2
        copy_in(i + 1, buf1, sem1)
        wait_in(buf0, sem0)
        accum(buf0)

        @pl.when(i + 2 < nch)
        def _():
            copy_in(i + 2, buf0, sem0)

        wait_in(buf1, sem1)
        accum(buf1)
        return 0

    lax.fori_loop(0, nch // 2, body, 0)
    pltpu.sync_copy(acc, out_ref.at[c, pl.ds(s, 1)])

    rows_h = hid_ref.shape[1] // _NS
    h0 = s * rows_h
    pltpu.sync_copy(hid_ref.at[c, pl.ds(h0, rows_h)],
                    ftok_ref.at[c, pl.ds(h0, rows_h)])


def _tc_pass1_body(hid_ref, ln1g_ref, ln1b_ref, w1_ref, b1_ref,
                   ln2g_ref, ln2b_ref, wd_ref, *rest, eps, tc_rows):
    if tc_rows:
        scores_ref, sums_ref, diff_ref = rest
    else:
        sums_ref, diff_ref = rest
    j = pl.program_id(1)

    if tc_rows:
        @pl.when(j == 0)
        def _init():
            sums_ref[...] = jnp.zeros_like(sums_ref)

        sums_ref[...] += jnp.sum(scores_ref[...], axis=1, keepdims=True)

    x = hid_ref[0]
    m = jnp.mean(x, axis=-1, keepdims=True)
    v = jnp.mean((x - m) ** 2, axis=-1, keepdims=True)
    x = (x - m) * lax.rsqrt(v + eps) * ln1g_ref[0] + ln1b_ref[0]

    h = jnp.dot(x, w1_ref[...], preferred_element_type=jnp.float32) + b1_ref[0]
    m = jnp.mean(h, axis=-1, keepdims=True)
    v = jnp.mean((h - m) ** 2, axis=-1, keepdims=True)
    h = (h - m) * lax.rsqrt(v + eps) * ln2g_ref[0] + ln2b_ref[0]
    h = 0.5 * h * (1.0 + lax.erf(h * (1.0 / math.sqrt(2.0))))

    diff_ref[...] = jnp.sum(h * wd_ref[0], axis=-1)[None, None, :]


def _tc_pass2_body(bd_ref, sc_ref, sums_ref, diff_ref, mask_ref,
                   *, n_rows, use_tc_sums):
    L = diff_ref.shape[2]
    idx = lax.broadcasted_iota(jnp.int32, (1, L), 1)

    total = jnp.sum(sc_ref[0], axis=0, keepdims=True)
    if use_tc_sums:
        total = total + sums_ref[0]
    imp = total * (1.0 / n_rows)
    mn = jnp.min(jnp.where(idx == 0, jnp.inf, imp))
    mx = jnp.max(jnp.where(idx == 0, -jnp.inf, imp))
    impn = (imp - mn) / mx
    impn = jnp.where(idx == 0, 1.0, impn)

    diff = diff_ref[0] + bd_ref[0]
    diff = jnp.where(idx == 0, diff + 100.0, diff)
    learned = jax.nn.sigmoid(diff)

    final = (impn + learned) * 0.5
    mask_ref[0] = (final >= 0.5).astype(mask_ref.dtype)


def kernel(hidden_states, attention_mask, self_attention_scores, key_layer,
           tome_size, ln1_g, ln1_b, W1, b1, ln2_g, ln2_b, W2, b2):
    B, L, D = hidden_states.shape
    H = self_attention_scores.shape[1]
    n_rows = H * L
    sc_rows = _SC_ROWS
    tc_rows = n_rows - sc_rows
    spw = sc_rows // _NS

    scores = self_attention_scores.reshape(B, n_rows, L)

    sc_part, final_token = pl.kernel(
        functools.partial(_sc_reduce_body, row_base=tc_rows, spw=spw,
                          ch=_CH, nl=L),
        out_type=(
            jax.ShapeDtypeStruct((_NC, _NS, L), jnp.float32),
            jax.ShapeDtypeStruct((B, L, D), hidden_states.dtype),
        ),
        mesh=plsc.VectorSubcoreMesh(core_axis_name="c", subcore_axis_name="s"),
        scratch_types=[
            pltpu.VMEM((_CH, L), jnp.float32),
            pltpu.VMEM((_CH, L), jnp.float32),
            pltpu.VMEM((1, L), jnp.float32),
            pltpu.SemaphoreType.DMA,
            pltpu.SemaphoreType.DMA,
        ],
    )(scores, hidden_states)

    nst = 16
    T = L // nst
    R = tc_rows // nst
    w_diff = W2[:, 0] - W2[:, 1]
    b_diff = (b2[0] - b2[1]).astype(jnp.float32)

    in_specs = [
        pl.BlockSpec((1, T, D), lambda b, j: (b, j, 0)),
        pl.BlockSpec((1, D), lambda b, j: (0, 0)),
        pl.BlockSpec((1, D), lambda b, j: (0, 0)),
        pl.BlockSpec((D, D), lambda b, j: (0, 0)),
        pl.BlockSpec((1, D), lambda b, j: (0, 0)),
        pl.BlockSpec((1, D), lambda b, j: (0, 0)),
        pl.BlockSpec((1, D), lambda b, j: (0, 0)),
        pl.BlockSpec((1, D), lambda b, j: (0, 0)),
    ]
    args = [hidden_states, ln1_g[None], ln1_b[None], W1, b1[None],
            ln2_g[None], ln2_b[None], w_diff[None]]
    if tc_rows:
        in_specs.append(pl.BlockSpec((1, R, L), lambda b, j: (b, j, 0)))
        args.append(scores)

    tc_sums, diff = pl.pallas_call(
        functools.partial(_tc_pass1_body, eps=1e-5, tc_rows=tc_rows),
        grid=(B, nst),
        in_specs=in_specs,
        out_specs=[
            pl.BlockSpec((1, 1, L), lambda b, j: (b, 0, 0)),
            pl.BlockSpec((1, 1, T), lambda b, j: (b, 0, j)),
        ],
        out_shape=[
            jax.ShapeDtypeStruct((B, 1, L), jnp.float32),
            jax.ShapeDtypeStruct((B, 1, L), jnp.float32),
        ],
        compiler_params=pltpu.CompilerParams(
            dimension_semantics=("parallel", "arbitrary")),
    )(*args)

    sc_view = sc_part.reshape(B, (_NC * _NS) // B, L)

    mask = pl.pallas_call(
        functools.partial(_tc_pass2_body, n_rows=float(n_rows),
                          use_tc_sums=bool(tc_rows)),
        grid=(B,),
        in_specs=[
            pl.BlockSpec(memory_space=pltpu.SMEM),
            pl.BlockSpec((1, (_NC * _NS) // B, L), lambda b: (b, 0, 0)),
            pl.BlockSpec((1, 1, L), lambda b: (b, 0, 0)),
            pl.BlockSpec((1, 1, L), lambda b: (b, 0, 0)),
        ],
        out_specs=pl.BlockSpec((1, 1, L), lambda b: (b, 0, 0)),
        out_shape=jax.ShapeDtypeStruct((B, 1, L), jnp.float32),
    )(b_diff[None], sc_view, tc_sums, diff)
    mask = mask.reshape(B, L)

    tome_size_new = jnp.ones((B, L, 1), dtype=attention_mask.dtype)
    return (final_token, attention_mask, tome_size_new, mask)


# speedup vs baseline: 2.8498x; 2.8498x over previous
import functools
import math

import jax
import jax.numpy as jnp
from jax import lax
from jax.experimental import pallas as pl
from jax.experimental.pallas import tpu as pltpu
from jax.experimental.pallas import tpu_sc as plsc

_NC = 2
_NS = 16

_SC_ROWS = 4608
_CH = 16


def _sc_reduce_body(scores_ref, out_ref, buf0, buf1, acc,
                    sem0, sem1, *, row_base, spw, ch, nl):
    c = lax.axis_index("c")
    s = lax.axis_index("s")
    row0 = row_base + s * spw
    nch = spw // ch
    zero16 = jnp.zeros((16,), jnp.float32)

    def zbody(j, _):
        acc[0, pl.ds(j * 16, 16)] = zero16
        return 0

    lax.fori_loop(0, nl // 16, zbody, 0)

    def copy_in(i, buf, sem):
        return pltpu.async_copy(
            scores_ref.at[c, pl.ds(row0 + i * ch, ch)], buf, sem)

    def wait_in(buf, sem):
        pltpu.make_async_copy(scores_ref.at[c, pl.ds(row0, ch)], buf, sem).wait()

    def accum(buf):
        def jbody(j, _):
            off = j * 16

            def rbody(r4, vs):
                v0, v1, v2, v3 = vs
                r = r4 * 4
                v0 = v0 + buf[r, pl.ds(off, 16)]
                v1 = v1 + buf[r + 1, pl.ds(off, 16)]
                v2 = v2 + buf[r + 2, pl.ds(off, 16)]
                v3 = v3 + buf[r + 3, pl.ds(off, 16)]
                return (v0, v1, v2, v3)

            v0, v1, v2, v3 = lax.fori_loop(
                0, ch // 4, rbody, (zero16, zero16, zero16, zero16), unroll=2)
            plsc.addupdate(acc.at[0, pl.ds(off, 16)], (v0 + v1) + (v2 + v3))
            return 0

        lax.fori_loop(0, nl // 16, jbody, 0, unroll=2)

    copy_in(0, buf0, sem0)

    def body(i2, _):
        i = i2 * 2
        copy_in(i + 1, buf1, sem1)
        wait_in(buf0, sem0)
        accum(buf0)

        @pl.when(i + 2 < nch)
        def _():
            copy_in(i + 2, buf0, sem0)

        wait_in(buf1, sem1)
        accum(buf1)
        return 0

    lax.fori_loop(0, nch // 2, body, 0)
    pltpu.sync_copy(acc, out_ref.at[c, pl.ds(s, 1)])


def _tc_pass1_body(hid_ref, ln1g_ref, ln1b_ref, w1_ref, b1_ref,
                   ln2g_ref, ln2b_ref, wd_ref, *rest, eps, tc_rows):
    if tc_rows:
        scores_ref, sums_ref, diff_ref = rest
    else:
        sums_ref, diff_ref = rest
    j = pl.program_id(1)

    if tc_rows:
        @pl.when(j == 0)
        def _init():
            sums_ref[...] = jnp.zeros_like(sums_ref)

        sums_ref[...] += jnp.sum(scores_ref[...], axis=1, keepdims=True)

    x = hid_ref[0]
    m = jnp.mean(x, axis=-1, keepdims=True)
    v = jnp.mean((x - m) ** 2, axis=-1, keepdims=True)
    x = (x - m) * lax.rsqrt(v + eps) * ln1g_ref[0] + ln1b_ref[0]

    h = jnp.dot(x, w1_ref[...], preferred_element_type=jnp.float32) + b1_ref[0]
    m = jnp.mean(h, axis=-1, keepdims=True)
    v = jnp.mean((h - m) ** 2, axis=-1, keepdims=True)
    h = (h - m) * lax.rsqrt(v + eps) * ln2g_ref[0] + ln2b_ref[0]
    h = 0.5 * h * (1.0 + lax.erf(h * (1.0 / math.sqrt(2.0))))

    diff_ref[...] = jnp.sum(h * wd_ref[0], axis=-1)[None, None, :]


def _tc_pass2_body(bd_ref, sc_ref, sums_ref, diff_ref, mask_ref,
                   *, n_rows, use_tc_sums):
    L = diff_ref.shape[2]
    idx = lax.broadcasted_iota(jnp.int32, (1, L), 1)

    total = jnp.sum(sc_ref[0], axis=0, keepdims=True)
    if use_tc_sums:
        total = total + sums_ref[0]
    imp = total * (1.0 / n_rows)
    mn = jnp.min(jnp.where(idx == 0, jnp.inf, imp))
    mx = jnp.max(jnp.where(idx == 0, -jnp.inf, imp))
    impn = (imp - mn) / mx
    impn = jnp.where(idx == 0, 1.0, impn)

    diff = diff_ref[0] + bd_ref[0]
    diff = jnp.where(idx == 0, diff + 100.0, diff)
    learned = jax.nn.sigmoid(diff)

    final = (impn + learned) * 0.5
    mask_ref[0] = (final >= 0.5).astype(mask_ref.dtype)


def kernel(hidden_states, attention_mask, self_attention_scores, key_layer,
           tome_size, ln1_g, ln1_b, W1, b1, ln2_g, ln2_b, W2, b2):
    B, L, D = hidden_states.shape
    H = self_attention_scores.shape[1]
    n_rows = H * L
    sc_rows = _SC_ROWS
    tc_rows = n_rows - sc_rows
    spw = sc_rows // _NS

    scores = self_attention_scores.reshape(B, n_rows, L)

    sc_part = pl.kernel(
        functools.partial(_sc_reduce_body, row_base=tc_rows, spw=spw,
                          ch=_CH, nl=L),
        out_type=jax.ShapeDtypeStruct((_NC, _NS, L), jnp.float32),
        mesh=plsc.VectorSubcoreMesh(core_axis_name="c", subcore_axis_name="s"),
        scratch_types=[
            pltpu.VMEM((_CH, L), jnp.float32),
            pltpu.VMEM((_CH, L), jnp.float32),
            pltpu.VMEM((1, L), jnp.float32),
            pltpu.SemaphoreType.DMA,
            pltpu.SemaphoreType.DMA,
        ],
    )(scores)

    nst = 16
    T = L // nst
    R = tc_rows // nst
    w_diff = W2[:, 0] - W2[:, 1]
    b_diff = (b2[0] - b2[1]).astype(jnp.float32)

    in_specs = [
        pl.BlockSpec((1, T, D), lambda b, j: (b, j, 0)),
        pl.BlockSpec((1, D), lambda b, j: (0, 0)),
        pl.BlockSpec((1, D), lambda b, j: (0, 0)),
        pl.BlockSpec((D, D), lambda b, j: (0, 0)),
        pl.BlockSpec((1, D), lambda b, j: (0, 0)),
        pl.BlockSpec((1, D), lambda b, j: (0, 0)),
        pl.BlockSpec((1, D), lambda b, j: (0, 0)),
        pl.BlockSpec((1, D), lambda b, j: (0, 0)),
    ]
    args = [hidden_states, ln1_g[None], ln1_b[None], W1, b1[None],
            ln2_g[None], ln2_b[None], w_diff[None]]
    if tc_rows:
        in_specs.append(pl.BlockSpec((1, R, L), lambda b, j: (b, j, 0)))
        args.append(scores)

    tc_sums, diff = pl.pallas_call(
        functools.partial(_tc_pass1_body, eps=1e-5, tc_rows=tc_rows),
        grid=(B, nst),
        in_specs=in_specs,
        out_specs=[
            pl.BlockSpec((1, 1, L), lambda b, j: (b, 0, 0)),
            pl.BlockSpec((1, 1, T), lambda b, j: (b, 0, j)),
        ],
        out_shape=[
            jax.ShapeDtypeStruct((B, 1, L), jnp.float32),
            jax.ShapeDtypeStruct((B, 1, L), jnp.float32),
        ],
        compiler_params=pltpu.CompilerParams(
            dimension_semantics=("parallel", "arbitrary")),
    )(*args)

    sc_view = sc_part.reshape(B, (_NC * _NS) // B, L)

    mask = pl.pallas_call(
        functools.partial(_tc_pass2_body, n_rows=float(n_rows),
                          use_tc_sums=bool(tc_rows)),
        grid=(B,),
        in_specs=[
            pl.BlockSpec(memory_space=pltpu.SMEM),
            pl.BlockSpec((1, (_NC * _NS) // B, L), lambda b: (b, 0, 0)),
            pl.BlockSpec((1, 1, L), lambda b: (b, 0, 0)),
            pl.BlockSpec((1, 1, L), lambda b: (b, 0, 0)),
        ],
        out_specs=pl.BlockSpec((1, 1, L), lambda b: (b, 0, 0)),
        out_shape=jax.ShapeDtypeStruct((B, 1, L), jnp.float32),
    )(b_diff[None], sc_view, tc_sums, diff)
    mask = mask.reshape(B, L)

    tome_size_new = jnp.ones((B, L, 1), dtype=attention_mask.dtype)
    return (hidden_states, attention_mask, tome_size_new, mask)
